# baseline (speedup 1.0000x reference)
import functools

import jax
import jax.numpy as jnp
from jax import lax
from jax.experimental import pallas as pl
from jax.experimental.pallas import tpu as pltpu
from jax.experimental.pallas import tpu_sc as plsc

N = 16384
D = 4
NUM_CORES = 2
NUM_SUBCORES = 16
NUM_WORKERS = NUM_CORES * NUM_SUBCORES
NBLK = N // 128
BLKS_PER_W = NBLK // NUM_WORKERS
ROWS_PER_W = N // NUM_WORKERS
VECS_PER_W = ROWS_PER_W // 16


def _sc_body(a_hbm, b_hbm, lbl_hbm, out_hbm,
             a_v, b_v, l_v, res_v, all_v, sa, sb, sl, shared):
    cid = lax.axis_index("c")
    sid = lax.axis_index("s")
    wid = sid * NUM_CORES + cid

    cp_a = pltpu.make_async_copy(
        a_hbm.at[pl.ds(wid * BLKS_PER_W, BLKS_PER_W)], a_v, sa)
    cp_b = pltpu.make_async_copy(
        b_hbm.at[pl.ds(wid * BLKS_PER_W, BLKS_PER_W)], b_v, sb)
    cp_l = pltpu.make_async_copy(
        lbl_hbm.at[pl.ds(wid * ROWS_PER_W, ROWS_PER_W)], l_v, sl)
    cp_a.start()
    cp_b.start()
    cp_l.start()
    cp_a.wait()
    cp_b.wait()
    cp_l.wait()

    def body(k, carry):
        a0, a1, a2, a3, cnt = carry
        blk = k >> 3
        s = pl.ds((k & 7) * 16, 16)
        mf = l_v[pl.ds(k * 16, 16)].astype(jnp.float32)
        d0 = a_v[blk, 0, s] - b_v[blk, 0, s]
        d1 = a_v[blk, 1, s] - b_v[blk, 1, s]
        d2 = a_v[blk, 2, s] - b_v[blk, 2, s]
        d3 = a_v[blk, 3, s] - b_v[blk, 3, s]
        return (a0 + d0 * d0 * mf, a1 + d1 * d1 * mf,
                a2 + d2 * d2 * mf, a3 + d3 * d3 * mf, cnt + mf)

    z = jnp.zeros((16,), jnp.float32)
    a0, a1, a2, a3, cnt = lax.fori_loop(0, VECS_PER_W, body, (z, z, z, z, z))
    acc = (a0 + a1) + (a2 + a3)

    res_v[0] = acc
    res_v[1] = cnt
    pltpu.sync_copy(res_v, shared.at[sid])
    plsc.subcore_barrier()

    @pl.when(sid == 0)
    def _():
        pltpu.sync_copy(shared, all_v)

        def red_body(i, carry):
            ts, tc = carry
            return ts + all_v[i, 0], tc + all_v[i, 1]

        ts, tc = lax.fori_loop(0, NUM_SUBCORES, red_body, (z, z))
        res_v[0] = jnp.full((16,), jnp.sum(ts), dtype=jnp.float32)
        res_v[1] = jnp.full((16,), jnp.sum(tc), dtype=jnp.float32)
        pltpu.sync_copy(res_v, out_hbm.at[cid])


@jax.jit
def _bbox_loss(a, b, label):
    mesh = plsc.VectorSubcoreMesh(core_axis_name="c", subcore_axis_name="s",
                                  num_cores=NUM_CORES)
    call = functools.partial(
        pl.kernel,
        out_type=jax.ShapeDtypeStruct((NUM_CORES, 2, 16), jnp.float32),
        mesh=mesh,
        compiler_params=pltpu.CompilerParams(needs_layout_passes=False,
                                             use_tc_tiling_on_sc=False,
                                             skip_device_barrier=True),
        scratch_types=[
            pltpu.VMEM((BLKS_PER_W, D, 128), jnp.float32),
            pltpu.VMEM((BLKS_PER_W, D, 128), jnp.float32),
            pltpu.VMEM((ROWS_PER_W,), jnp.int32),
            pltpu.VMEM((2, 16), jnp.float32),
            pltpu.VMEM((NUM_SUBCORES, 2, 16), jnp.float32),
            pltpu.SemaphoreType.DMA,
            pltpu.SemaphoreType.DMA,
            pltpu.SemaphoreType.DMA,
            pltpu.VMEM_SHARED((NUM_SUBCORES, 2, 16), jnp.float32),
        ],
    )(_sc_body)
    out = call(a, b, label)
    return out[0, 0, 0] + out[1, 0, 0], out[0, 1, 0] + out[1, 1, 0]


def kernel(bbox_out, bbox_target, label):
    a = bbox_out.reshape(NBLK, 128, D).swapaxes(1, 2)
    b = bbox_target.reshape(NBLK, 128, D).swapaxes(1, 2)
    s, c = _bbox_loss(a, b, label)
    return s / c


# speedup vs baseline: 1.2848x; 1.2848x over previous
import functools

import jax
import jax.numpy as jnp
from jax import lax
from jax.experimental import pallas as pl
from jax.experimental.pallas import tpu as pltpu
from jax.experimental.pallas import tpu_sc as plsc

N = 16384
D = 4
NUM_TILES = 16
ROWS_PER_TILE = N // NUM_TILES
VECS_PER_TILE = ROWS_PER_TILE // 16
NBLK = N // 128
BLKS_PER_TILE = NBLK // NUM_TILES


def _sc_body(a_hbm, b_hbm, lbl_hbm, out_hbm,
             a_v, b_v, l_v, res_v, all_v, out_v, sa, sb, sl, shared):
    sid = lax.axis_index("s")
    blk0 = sid * BLKS_PER_TILE

    cp_a = pltpu.make_async_copy(a_hbm.at[pl.ds(blk0, BLKS_PER_TILE)], a_v, sa)
    cp_b = pltpu.make_async_copy(b_hbm.at[pl.ds(blk0, BLKS_PER_TILE)], b_v, sb)
    cp_l = pltpu.make_async_copy(
        lbl_hbm.at[pl.ds(sid * ROWS_PER_TILE, ROWS_PER_TILE)], l_v, sl)
    cp_a.start()
    cp_b.start()
    cp_l.start()
    cp_a.wait()
    cp_b.wait()
    cp_l.wait()

    def body(k, carry):
        a0, a1, a2, a3, cnt = carry
        blk = k >> 3
        s = pl.ds((k & 7) * 16, 16)
        mf = l_v[pl.ds(k * 16, 16)].astype(jnp.float32)
        d0 = a_v[blk, 0, s] - b_v[blk, 0, s]
        d1 = a_v[blk, 1, s] - b_v[blk, 1, s]
        d2 = a_v[blk, 2, s] - b_v[blk, 2, s]
        d3 = a_v[blk, 3, s] - b_v[blk, 3, s]
        return (a0 + d0 * d0 * mf, a1 + d1 * d1 * mf,
                a2 + d2 * d2 * mf, a3 + d3 * d3 * mf, cnt + mf)

    z = jnp.zeros((16,), jnp.float32)
    a0, a1, a2, a3, cnt = lax.fori_loop(0, VECS_PER_TILE, body,
                                        (z, z, z, z, z))
    acc = (a0 + a1) + (a2 + a3)

    res_v[0] = acc
    res_v[1] = cnt
    pltpu.sync_copy(res_v, shared.at[sid])
    plsc.subcore_barrier()

    @pl.when(sid == 0)
    def _():
        pltpu.sync_copy(shared, all_v)

        def red_body(i, carry):
            ts, tc = carry
            return ts + all_v[i, 0], tc + all_v[i, 1]

        ts, tc = lax.fori_loop(0, NUM_TILES, red_body, (z, z))
        s_vec = jnp.full((16,), jnp.sum(ts), dtype=jnp.float32)
        c_vec = jnp.full((16,), jnp.sum(tc), dtype=jnp.float32)
        out_v[...] = s_vec / c_vec
        pltpu.sync_copy(out_v, out_hbm)


@jax.jit
def _bbox_loss(a, b, label):
    mesh = plsc.VectorSubcoreMesh(core_axis_name="c", subcore_axis_name="s",
                                  num_cores=1)
    call = functools.partial(
        pl.kernel,
        out_type=jax.ShapeDtypeStruct((16,), jnp.float32),
        mesh=mesh,
        compiler_params=pltpu.CompilerParams(needs_layout_passes=False,
                                             use_tc_tiling_on_sc=False,
                                             skip_device_barrier=True),
        scratch_types=[
            pltpu.VMEM((BLKS_PER_TILE, D, 128), jnp.float32),
            pltpu.VMEM((BLKS_PER_TILE, D, 128), jnp.float32),
            pltpu.VMEM((ROWS_PER_TILE,), jnp.int32),
            pltpu.VMEM((2, 16), jnp.float32),
            pltpu.VMEM((NUM_TILES, 2, 16), jnp.float32),
            pltpu.VMEM((16,), jnp.float32),
            pltpu.SemaphoreType.DMA,
            pltpu.SemaphoreType.DMA,
            pltpu.SemaphoreType.DMA,
            pltpu.VMEM_SHARED((NUM_TILES, 2, 16), jnp.float32),
        ],
    )(_sc_body)
    out = call(a, b, label)
    return out[0]


def kernel(bbox_out, bbox_target, label):
    a = bbox_out.reshape(NBLK, 128, D).swapaxes(1, 2)
    b = bbox_target.reshape(NBLK, 128, D).swapaxes(1, 2)
    return _bbox_loss(a, b, label)
